# baseline (speedup 1.0000x reference)
import functools

import jax
import jax.numpy as jnp
from jax import lax
from jax.experimental import pallas as pl
from jax.experimental.pallas import tpu as pltpu
from jax.experimental.pallas import tpu_sc as plsc

D = 64
NC = 2
NS = 16
NW = NC * NS
C = 128
NBUF = 5


def _build_gather(n_rows):
    n_per_w = n_rows // NW
    n_chunks = n_per_w // C
    n_groups = n_chunks // NBUF
    assert n_per_w * NW == n_rows and n_chunks * C == n_per_w
    assert n_groups * NBUF == n_chunks

    mesh = plsc.VectorSubcoreMesh(core_axis_name="c", subcore_axis_name="s")

    @functools.partial(
        pl.kernel,
        mesh=mesh,
        out_type=jax.ShapeDtypeStruct((n_rows, D), jnp.float32),
        scratch_types=(
            [pltpu.VMEM((n_chunks, C), jnp.int32)]
            + [pltpu.VMEM((C, D), jnp.float32) for _ in range(NBUF)]
            + [pltpu.SemaphoreType.DMA for _ in range(2 * NBUF)]
        ),
        compiler_params=pltpu.CompilerParams(use_tc_tiling_on_sc=False),
    )
    def gather(ids_hbm, table_hbm, out_hbm, idx_v, *rest):
        bufs = rest[:NBUF]
        gsems = rest[NBUF : 2 * NBUF]
        osems = rest[2 * NBUF :]

        wid = lax.axis_index("s") * NC + lax.axis_index("c")
        base = wid * n_per_w

        pltpu.sync_copy(ids_hbm.at[wid], idx_v)

        for b in range(NBUF):
            pltpu.async_copy(table_hbm.at[idx_v.at[b]], bufs[b], gsems[b])

        @pl.loop(0, n_groups)
        def _(g):
            for b in range(NBUF):
                j = g * NBUF + b
                pltpu.make_async_copy(
                    table_hbm.at[idx_v.at[j]], bufs[b], gsems[b]
                ).wait()
                pltpu.async_copy(
                    bufs[b], out_hbm.at[pl.ds(base + j * C, C)], osems[b]
                )

                @pl.when(g < n_groups - 1)
                def _():
                    pltpu.make_async_copy(
                        bufs[b], out_hbm.at[pl.ds(base, C)], osems[b]
                    ).wait()
                    pltpu.async_copy(
                        table_hbm.at[idx_v.at[j + NBUF]], bufs[b], gsems[b]
                    )

        for b in range(NBUF):
            pltpu.make_async_copy(
                bufs[b], out_hbm.at[pl.ds(base, C)], osems[b]
            ).wait()

    return gather


def kernel(word_ids, emb_wt):
    n_rows = word_ids.shape[0] * word_ids.shape[1]
    n_per_w = n_rows // NW
    ids = word_ids.astype(jnp.int32).reshape(NW, n_per_w // C, C)
    out = _build_gather(n_rows)(ids, emb_wt)
    return out.reshape(*word_ids.shape, D)


# speedup vs baseline: 1.0028x; 1.0028x over previous
import functools

import jax
import jax.numpy as jnp
from jax import lax
from jax.experimental import pallas as pl
from jax.experimental.pallas import tpu as pltpu
from jax.experimental.pallas import tpu_sc as plsc

D = 64
NC = 2
NS = 16
NW = NC * NS
C = 128
NBUF = 5


def _build_gather(n_rows):
    n_per_w = n_rows // NW
    n_chunks = n_per_w // C
    n_groups = n_chunks // NBUF
    assert n_per_w * NW == n_rows and n_chunks * C == n_per_w
    assert n_groups * NBUF == n_chunks

    mesh = plsc.VectorSubcoreMesh(core_axis_name="c", subcore_axis_name="s")

    @functools.partial(
        pl.kernel,
        mesh=mesh,
        out_type=jax.ShapeDtypeStruct((n_rows, D), jnp.float32),
        scratch_types=(
            [pltpu.VMEM((n_chunks, C), jnp.int32)]
            + [pltpu.VMEM((C, D), jnp.float32) for _ in range(NBUF)]
            + [pltpu.SemaphoreType.DMA for _ in range(2 * NBUF)]
        ),
        compiler_params=pltpu.CompilerParams(use_tc_tiling_on_sc=False),
    )
    def gather(ids_hbm, table_hbm, out_hbm, idx_v, *rest):
        bufs = rest[:NBUF]
        gsems = rest[NBUF : 2 * NBUF]
        osems = rest[2 * NBUF :]

        wid = lax.axis_index("s") * NC + lax.axis_index("c")
        base = wid * n_per_w

        pltpu.sync_copy(ids_hbm.at[pl.ds(wid * n_chunks, n_chunks)], idx_v)

        for b in range(NBUF):
            pltpu.async_copy(table_hbm.at[idx_v.at[b]], bufs[b], gsems[b])

        @pl.loop(0, n_groups)
        def _(g):
            for b in range(NBUF):
                j = g * NBUF + b
                pltpu.make_async_copy(
                    table_hbm.at[idx_v.at[j]], bufs[b], gsems[b]
                ).wait()
                pltpu.async_copy(
                    bufs[b], out_hbm.at[pl.ds(base + j * C, C)], osems[b]
                )

                @pl.when(g < n_groups - 1)
                def _():
                    pltpu.make_async_copy(
                        bufs[b], out_hbm.at[pl.ds(base, C)], osems[b]
                    ).wait()
                    pltpu.async_copy(
                        table_hbm.at[idx_v.at[j + NBUF]], bufs[b], gsems[b]
                    )

        for b in range(NBUF):
            pltpu.make_async_copy(
                bufs[b], out_hbm.at[pl.ds(base, C)], osems[b]
            ).wait()

    return gather


def kernel(word_ids, emb_wt):
    n_rows = word_ids.shape[0] * word_ids.shape[1]
    ids = word_ids.astype(jnp.int32).reshape(n_rows // C, C)
    tab = jax.lax.optimization_barrier(emb_wt.reshape(-1, 2 * D))
    tab = tab.reshape(-1, D)
    out = _build_gather(n_rows)(ids, tab)
    return out.reshape(*word_ids.shape, D)


# speedup vs baseline: 1.0579x; 1.0549x over previous
import functools

import jax
import jax.numpy as jnp
from jax import lax
from jax.experimental import pallas as pl
from jax.experimental.pallas import tpu as pltpu
from jax.experimental.pallas import tpu_sc as plsc

D = 64
NC = 2
NS = 16
NW = NC * NS
C = 128
NBUF = 5


def _build_gather(n_rows):
    n_per_w = n_rows // NW
    n_chunks = n_per_w // C
    n_groups = n_chunks // NBUF
    assert n_per_w * NW == n_rows and n_chunks * C == n_per_w
    assert n_groups * NBUF == n_chunks

    mesh = plsc.VectorSubcoreMesh(core_axis_name="c", subcore_axis_name="s")

    @functools.partial(
        pl.kernel,
        mesh=mesh,
        out_type=jax.ShapeDtypeStruct((n_rows, D), jnp.float32),
        scratch_types=(
            [pltpu.VMEM((n_chunks, C), jnp.int32)]
            + [pltpu.VMEM((C, 2 * D), jnp.float32) for _ in range(NBUF)]
            + [pltpu.SemaphoreType.DMA for _ in range(2 * NBUF)]
        ),
        compiler_params=pltpu.CompilerParams(use_tc_tiling_on_sc=False),
    )
    def gather(ids_hbm, table_hbm, out_hbm, idx_v, *rest):
        bufs = rest[:NBUF]
        gsems = rest[NBUF : 2 * NBUF]
        osems = rest[2 * NBUF :]

        wid = lax.axis_index("s") * NC + lax.axis_index("c")
        base = wid * n_per_w

        pltpu.sync_copy(ids_hbm.at[pl.ds(wid * n_chunks, n_chunks)], idx_v)

        for b in range(NBUF):
            pltpu.async_copy(table_hbm.at[idx_v.at[b]], bufs[b], gsems[b])

        @pl.loop(0, n_groups)
        def _(g):
            for b in range(NBUF):
                j = g * NBUF + b
                pltpu.make_async_copy(
                    table_hbm.at[idx_v.at[j]], bufs[b], gsems[b]
                ).wait()
                pltpu.async_copy(
                    bufs[b].at[:, pl.ds(0, D)],
                    out_hbm.at[pl.ds(base + j * C, C)],
                    osems[b],
                )

                @pl.when(g < n_groups - 1)
                def _():
                    pltpu.make_async_copy(
                        bufs[b].at[:, pl.ds(0, D)],
                        out_hbm.at[pl.ds(base, C)],
                        osems[b],
                    ).wait()
                    pltpu.async_copy(
                        table_hbm.at[idx_v.at[j + NBUF]], bufs[b], gsems[b]
                    )

        for b in range(NBUF):
            pltpu.make_async_copy(
                bufs[b].at[:, pl.ds(0, D)], out_hbm.at[pl.ds(base, C)], osems[b]
            ).wait()

    return gather


def kernel(word_ids, emb_wt):
    n_rows = word_ids.shape[0] * word_ids.shape[1]
    ids = word_ids.astype(jnp.int32).reshape(n_rows // C, C)
    tab = jnp.pad(emb_wt, ((0, 0), (0, D)))
    out = _build_gather(n_rows)(ids, tab)
    return out.reshape(*word_ids.shape, D)


# speedup vs baseline: 1.3015x; 1.2303x over previous
import functools

import jax
import jax.numpy as jnp
from jax import lax
from jax.experimental import pallas as pl
from jax.experimental.pallas import tpu as pltpu
from jax.experimental.pallas import tpu_sc as plsc

D = 64
NC = 2
NS = 16
NW = NC * NS
C = 128
NBUF = 5


def _build_gather(n_rows):
    n_per_w = n_rows // NW
    n_chunks = n_per_w // C
    n_groups = n_chunks // NBUF
    assert n_per_w * NW == n_rows and n_chunks * C == n_per_w
    assert n_groups * NBUF == n_chunks

    mesh = plsc.VectorSubcoreMesh(core_axis_name="c", subcore_axis_name="s")

    @functools.partial(
        pl.kernel,
        mesh=mesh,
        out_type=jax.ShapeDtypeStruct((n_rows, D), jnp.float32),
        scratch_types=(
            [pltpu.VMEM((n_chunks, C), jnp.int32)]
            + [pltpu.VMEM((C, 2 * D), jnp.float32) for _ in range(NBUF)]
            + [pltpu.SemaphoreType.DMA for _ in range(2 * NBUF)]
        ),
        compiler_params=pltpu.CompilerParams(use_tc_tiling_on_sc=False),
    )
    def gather(ids_hbm, table_hbm, out_hbm, idx_v, *rest):
        bufs = rest[:NBUF]
        gsems = rest[NBUF : 2 * NBUF]
        osems = rest[2 * NBUF :]

        wid = lax.axis_index("s") * NC + lax.axis_index("c")
        base = wid * n_per_w

        pltpu.sync_copy(ids_hbm.at[pl.ds(wid * n_chunks, n_chunks)], idx_v)

        for b in range(NBUF):
            pltpu.async_copy(table_hbm.at[idx_v.at[b]], bufs[b], gsems[b])

        @pl.loop(0, n_groups)
        def _(g):
            for b in range(NBUF):
                j = g * NBUF + b
                pltpu.make_async_copy(
                    table_hbm.at[idx_v.at[j]], bufs[b], gsems[b]
                ).wait()
                pltpu.async_copy(
                    bufs[b].at[:, pl.ds(0, D)],
                    out_hbm.at[pl.ds(base + j * C, C)],
                    osems[b],
                )

                @pl.when(g < n_groups - 1)
                def _():
                    pltpu.make_async_copy(
                        bufs[b].at[:, pl.ds(0, D)],
                        out_hbm.at[pl.ds(base, C)],
                        osems[b],
                    ).wait()
                    pltpu.async_copy(
                        table_hbm.at[idx_v.at[j + NBUF]], bufs[b], gsems[b]
                    )

        for b in range(NBUF):
            pltpu.make_async_copy(
                bufs[b].at[:, pl.ds(0, D)], out_hbm.at[pl.ds(base, C)], osems[b]
            ).wait()

    return gather


TBLK = 4096


def _widen_table(emb_wt):
    tab_t = emb_wt.T
    rows = tab_t.shape[1]

    def body(x_ref, o_ref):
        y = x_ref[...].T
        o_ref[...] = jnp.concatenate([y, y], axis=1)

    return pl.pallas_call(
        body,
        grid=(pl.cdiv(rows, TBLK),),
        in_specs=[pl.BlockSpec((D, TBLK), lambda i: (0, i))],
        out_specs=pl.BlockSpec((TBLK, 2 * D), lambda i: (i, 0)),
        out_shape=jax.ShapeDtypeStruct((rows, 2 * D), jnp.float32),
    )(tab_t)


def kernel(word_ids, emb_wt):
    n_rows = word_ids.shape[0] * word_ids.shape[1]
    ids = word_ids.astype(jnp.int32).reshape(n_rows // C, C)
    tab = _widen_table(emb_wt)
    out = _build_gather(n_rows)(ids, tab)
    return out.reshape(*word_ids.shape, D)


# speedup vs baseline: 1.4018x; 1.0771x over previous
import functools

import jax
import jax.numpy as jnp
from jax import lax
from jax.experimental import pallas as pl
from jax.experimental.pallas import tpu as pltpu
from jax.experimental.pallas import tpu_sc as plsc

D = 64
NC = 2
NS = 16
NW = NC * NS
C = 128
NBUF = 5


def _build_gather(n_rows):
    n_per_w = n_rows // NW
    n_chunks = n_per_w // C
    n_groups = n_chunks // NBUF
    assert n_per_w * NW == n_rows and n_chunks * C == n_per_w
    assert n_groups * NBUF == n_chunks

    mesh = plsc.VectorSubcoreMesh(core_axis_name="c", subcore_axis_name="s")

    @functools.partial(
        pl.kernel,
        mesh=mesh,
        out_type=jax.ShapeDtypeStruct((n_rows, D), jnp.float32),
        scratch_types=(
            [pltpu.VMEM((n_chunks, C), jnp.int32)]
            + [pltpu.VMEM((C, 2 * D), jnp.float32) for _ in range(NBUF)]
            + [pltpu.SemaphoreType.DMA for _ in range(2 * NBUF)]
        ),
        compiler_params=pltpu.CompilerParams(use_tc_tiling_on_sc=False),
    )
    def gather(ids_hbm, table_hbm, out_hbm, idx_v, *rest):
        bufs = rest[:NBUF]
        gsems = rest[NBUF : 2 * NBUF]
        osems = rest[2 * NBUF :]

        wid = lax.axis_index("s") * NC + lax.axis_index("c")
        base = wid * n_per_w

        pltpu.sync_copy(ids_hbm.at[pl.ds(wid * n_chunks, n_chunks)], idx_v)

        for b in range(NBUF):
            pltpu.async_copy(table_hbm.at[idx_v.at[b]], bufs[b], gsems[b])

        @pl.loop(0, n_groups)
        def _(g):
            for b in range(NBUF):
                j = g * NBUF + b
                pltpu.make_async_copy(
                    table_hbm.at[idx_v.at[j]], bufs[b], gsems[b]
                ).wait()
                pltpu.async_copy(
                    bufs[b].at[:, pl.ds(0, D)],
                    out_hbm.at[pl.ds(base + j * C, C)],
                    osems[b],
                )

                @pl.when(g < n_groups - 1)
                def _():
                    pltpu.make_async_copy(
                        bufs[b].at[:, pl.ds(0, D)],
                        out_hbm.at[pl.ds(base, C)],
                        osems[b],
                    ).wait()
                    pltpu.async_copy(
                        table_hbm.at[idx_v.at[j + NBUF]], bufs[b], gsems[b]
                    )

        for b in range(NBUF):
            pltpu.make_async_copy(
                bufs[b].at[:, pl.ds(0, D)], out_hbm.at[pl.ds(base, C)], osems[b]
            ).wait()

    return gather


TBLK = 4096


def _widen_table(emb_wt):
    tab_t = emb_wt.T
    rows = tab_t.shape[1]

    def body(x_ref, o_ref):
        i = jax.lax.broadcasted_iota(jnp.int32, (D, 2 * D), 0)
        j = jax.lax.broadcasted_iota(jnp.int32, (D, 2 * D), 1)
        eye2 = (i == j % D).astype(jnp.float32)
        o_ref[...] = jax.lax.dot_general(
            x_ref[...],
            eye2,
            (((0,), (0,)), ((), ())),
            preferred_element_type=jnp.float32,
        )

    return pl.pallas_call(
        body,
        grid=(pl.cdiv(rows, TBLK),),
        in_specs=[pl.BlockSpec((D, TBLK), lambda i: (0, i))],
        out_specs=pl.BlockSpec((TBLK, 2 * D), lambda i: (i, 0)),
        out_shape=jax.ShapeDtypeStruct((rows, 2 * D), jnp.float32),
    )(tab_t)


def kernel(word_ids, emb_wt):
    n_rows = word_ids.shape[0] * word_ids.shape[1]
    ids = word_ids.astype(jnp.int32).reshape(n_rows // C, C)
    tab = _widen_table(emb_wt)
    out = _build_gather(n_rows)(ids, tab)
    return out.reshape(*word_ids.shape, D)


# speedup vs baseline: 1.7199x; 1.2269x over previous
import functools

import jax
import jax.numpy as jnp
from jax import lax
from jax.experimental import pallas as pl
from jax.experimental.pallas import tpu as pltpu
from jax.experimental.pallas import tpu_sc as plsc

D = 64
NC = 2
NS = 16
NW = NC * NS
C = 128
NBUF = 5


def _build_gather(n_rows):
    n_per_w = n_rows // NW
    n_chunks = n_per_w // C
    n_groups = n_chunks // NBUF
    assert n_per_w * NW == n_rows and n_chunks * C == n_per_w
    assert n_groups * NBUF == n_chunks

    mesh = plsc.VectorSubcoreMesh(core_axis_name="c", subcore_axis_name="s")

    @functools.partial(
        pl.kernel,
        mesh=mesh,
        out_type=jax.ShapeDtypeStruct((n_rows, D), jnp.float32),
        scratch_types=(
            [pltpu.VMEM((n_chunks, C), jnp.int32)]
            + [pltpu.VMEM((C, 2 * D), jnp.float32) for _ in range(NBUF)]
            + [pltpu.SemaphoreType.DMA for _ in range(2 * NBUF)]
        ),
        compiler_params=pltpu.CompilerParams(use_tc_tiling_on_sc=False),
    )
    def gather(ids_hbm, table_hbm, out_hbm, idx_v, *rest):
        bufs = rest[:NBUF]
        gsems = rest[NBUF : 2 * NBUF]
        osems = rest[2 * NBUF :]

        wid = lax.axis_index("s") * NC + lax.axis_index("c")
        base = wid * n_per_w

        pltpu.sync_copy(ids_hbm.at[pl.ds(wid * n_chunks, n_chunks)], idx_v)

        for b in range(NBUF):
            pltpu.async_copy(table_hbm.at[idx_v.at[b]], bufs[b], gsems[b])

        @pl.loop(0, n_groups)
        def _(g):
            for b in range(NBUF):
                j = g * NBUF + b
                pltpu.make_async_copy(
                    table_hbm.at[idx_v.at[j]], bufs[b], gsems[b]
                ).wait()
                pltpu.async_copy(
                    bufs[b].at[:, pl.ds(0, D)],
                    out_hbm.at[pl.ds(base + j * C, C)],
                    osems[b],
                )

                @pl.when(g < n_groups - 1)
                def _():
                    pltpu.make_async_copy(
                        bufs[b].at[:, pl.ds(0, D)],
                        out_hbm.at[pl.ds(base, C)],
                        osems[b],
                    ).wait()
                    pltpu.async_copy(
                        table_hbm.at[idx_v.at[j + NBUF]], bufs[b], gsems[b]
                    )

        for b in range(NBUF):
            pltpu.make_async_copy(
                bufs[b].at[:, pl.ds(0, D)], out_hbm.at[pl.ds(base, C)], osems[b]
            ).wait()

    return gather


TBLK = 16384


def _widen_table(emb_wt):
    tab_t = emb_wt.T
    rows = tab_t.shape[1]

    def body(x_ref, o_ref):
        i = jax.lax.broadcasted_iota(jnp.int32, (D, 2 * D), 0)
        j = jax.lax.broadcasted_iota(jnp.int32, (D, 2 * D), 1)
        eye2 = (i == j % D).astype(jnp.float32)
        o_ref[...] = jax.lax.dot_general(
            x_ref[...],
            eye2,
            (((0,), (0,)), ((), ())),
            preferred_element_type=jnp.float32,
        )

    return pl.pallas_call(
        body,
        grid=(pl.cdiv(rows, TBLK),),
        in_specs=[pl.BlockSpec((D, TBLK), lambda i: (0, i))],
        out_specs=pl.BlockSpec((TBLK, 2 * D), lambda i: (i, 0)),
        out_shape=jax.ShapeDtypeStruct((rows, 2 * D), jnp.float32),
    )(tab_t)


def kernel(word_ids, emb_wt):
    n_rows = word_ids.shape[0] * word_ids.shape[1]
    ids = word_ids.astype(jnp.int32).reshape(n_rows // C, C)
    tab = _widen_table(emb_wt)
    out = _build_gather(n_rows)(ids, tab)
    return out.reshape(*word_ids.shape, D)


# speedup vs baseline: 1.7459x; 1.0151x over previous
import functools

import jax
import jax.numpy as jnp
from jax import lax
from jax.experimental import pallas as pl
from jax.experimental.pallas import tpu as pltpu
from jax.experimental.pallas import tpu_sc as plsc

D = 64
NC = 2
NS = 16
NW = NC * NS
C = 128
NBUF = 5


def _build_gather(n_rows):
    n_per_w = n_rows // NW
    n_chunks = n_per_w // C
    n_groups = n_chunks // NBUF
    assert n_per_w * NW == n_rows and n_chunks * C == n_per_w
    assert n_groups * NBUF == n_chunks

    mesh = plsc.VectorSubcoreMesh(core_axis_name="c", subcore_axis_name="s")

    @functools.partial(
        pl.kernel,
        mesh=mesh,
        out_type=jax.ShapeDtypeStruct((n_rows, D), jnp.float32),
        scratch_types=(
            [pltpu.VMEM((n_chunks, C), jnp.int32)]
            + [pltpu.VMEM((C, 2 * D), jnp.float32) for _ in range(NBUF)]
            + [pltpu.SemaphoreType.DMA for _ in range(2 * NBUF)]
        ),
        compiler_params=pltpu.CompilerParams(use_tc_tiling_on_sc=False),
    )
    def gather(ids_hbm, table_hbm, out_hbm, idx_v, *rest):
        bufs = rest[:NBUF]
        gsems = rest[NBUF : 2 * NBUF]
        osems = rest[2 * NBUF :]

        wid = lax.axis_index("s") * NC + lax.axis_index("c")
        base = wid * n_per_w

        pltpu.sync_copy(ids_hbm.at[pl.ds(wid * n_chunks, n_chunks)], idx_v)

        for b in range(NBUF):
            pltpu.async_copy(table_hbm.at[idx_v.at[b]], bufs[b], gsems[b])

        @pl.loop(0, n_groups)
        def _(g):
            for b in range(NBUF):
                j = g * NBUF + b
                pltpu.make_async_copy(
                    table_hbm.at[idx_v.at[j]], bufs[b], gsems[b]
                ).wait()
                pltpu.async_copy(
                    bufs[b].at[:, pl.ds(0, D)],
                    out_hbm.at[pl.ds(base + j * C, C)],
                    osems[b],
                )

                @pl.when(g < n_groups - 1)
                def _():
                    pltpu.make_async_copy(
                        bufs[b].at[:, pl.ds(0, D)],
                        out_hbm.at[pl.ds(base, C)],
                        osems[b],
                    ).wait()
                    pltpu.async_copy(
                        table_hbm.at[idx_v.at[j + NBUF]], bufs[b], gsems[b]
                    )

        for b in range(NBUF):
            pltpu.make_async_copy(
                bufs[b].at[:, pl.ds(0, D)], out_hbm.at[pl.ds(base, C)], osems[b]
            ).wait()

    return gather


TBLK = 32768


def _widen_table(emb_wt):
    tab_t = emb_wt.T
    rows = tab_t.shape[1]

    def body(x_ref, o_ref):
        i = jax.lax.broadcasted_iota(jnp.int32, (D, 2 * D), 0)
        j = jax.lax.broadcasted_iota(jnp.int32, (D, 2 * D), 1)
        eye2 = (i == j % D).astype(jnp.float32)
        o_ref[...] = jax.lax.dot_general(
            x_ref[...],
            eye2,
            (((0,), (0,)), ((), ())),
            preferred_element_type=jnp.float32,
        )

    return pl.pallas_call(
        body,
        grid=(pl.cdiv(rows, TBLK),),
        in_specs=[pl.BlockSpec((D, TBLK), lambda i: (0, i))],
        out_specs=pl.BlockSpec((TBLK, 2 * D), lambda i: (i, 0)),
        out_shape=jax.ShapeDtypeStruct((rows, 2 * D), jnp.float32),
    )(tab_t)


def kernel(word_ids, emb_wt):
    n_rows = word_ids.shape[0] * word_ids.shape[1]
    ids = word_ids.astype(jnp.int32).reshape(n_rows // C, C)
    tab = _widen_table(emb_wt)
    out = _build_gather(n_rows)(ids, tab)
    return out.reshape(*word_ids.shape, D)


# speedup vs baseline: 1.8948x; 1.0853x over previous
import functools

import jax
import jax.numpy as jnp
from jax import lax
from jax.experimental import pallas as pl
from jax.experimental.pallas import tpu as pltpu
from jax.experimental.pallas import tpu_sc as plsc

D = 64
NC = 2
NS = 16
NW = NC * NS
C = 128
NBUF = 5


def _build_gather(n_rows):
    n_per_w = n_rows // NW
    n_chunks = n_per_w // C
    n_groups = n_chunks // NBUF
    assert n_per_w * NW == n_rows and n_chunks * C == n_per_w
    assert n_groups * NBUF == n_chunks

    mesh = plsc.VectorSubcoreMesh(core_axis_name="c", subcore_axis_name="s")

    @functools.partial(
        pl.kernel,
        mesh=mesh,
        out_type=jax.ShapeDtypeStruct((n_rows, D), jnp.float32),
        scratch_types=(
            [pltpu.VMEM((n_chunks, C), jnp.int32)]
            + [pltpu.VMEM((C, D), jnp.float32) for _ in range(NBUF)]
            + [pltpu.SemaphoreType.DMA for _ in range(2 * NBUF)]
        ),
        compiler_params=pltpu.CompilerParams(use_tc_tiling_on_sc=False),
    )
    def gather(ids_hbm, table_hbm, out_hbm, idx_v, *rest):
        bufs = rest[:NBUF]
        gsems = rest[NBUF : 2 * NBUF]
        osems = rest[2 * NBUF :]

        wid = lax.axis_index("s") * NC + lax.axis_index("c")
        base = wid * n_per_w

        pltpu.sync_copy(ids_hbm.at[pl.ds(wid * n_chunks, n_chunks)], idx_v)

        for b in range(NBUF):
            pltpu.async_copy(table_hbm.at[idx_v.at[b]], bufs[b], gsems[b])

        @pl.loop(0, n_groups)
        def _(g):
            for b in range(NBUF):
                j = g * NBUF + b
                pltpu.make_async_copy(
                    table_hbm.at[idx_v.at[j]], bufs[b], gsems[b]
                ).wait()
                pltpu.async_copy(
                    bufs[b], out_hbm.at[pl.ds(base + j * C, C)], osems[b]
                )

                @pl.when(g < n_groups - 1)
                def _():
                    pltpu.make_async_copy(
                        bufs[b], out_hbm.at[pl.ds(base, C)], osems[b]
                    ).wait()
                    pltpu.async_copy(
                        table_hbm.at[idx_v.at[j + NBUF]], bufs[b], gsems[b]
                    )

        for b in range(NBUF):
            pltpu.make_async_copy(
                bufs[b], out_hbm.at[pl.ds(base, C)], osems[b]
            ).wait()

    return gather


T2 = 8192
T2_LOG = T2.bit_length() - 1


def _pair_table(emb_wt):
    tab_t = emb_wt.T
    rows = tab_t.shape[1]
    nsb = pl.cdiv(rows, 2 * T2)

    def body(x_ref, o_ref):
        i = jax.lax.broadcasted_iota(jnp.int32, (D, D), 0)
        j = jax.lax.broadcasted_iota(jnp.int32, (D, D), 1)
        eye = (i == j).astype(jnp.float32)
        dn = (((0,), (0,)), ((), ()))
        o_ref[:, :D] = jax.lax.dot_general(
            x_ref[:, :T2], eye, dn, preferred_element_type=jnp.float32
        )
        o_ref[:, D:] = jax.lax.dot_general(
            x_ref[:, T2:], eye, dn, preferred_element_type=jnp.float32
        )

    return pl.pallas_call(
        body,
        grid=(nsb,),
        in_specs=[pl.BlockSpec((D, 2 * T2), lambda i: (0, i))],
        out_specs=pl.BlockSpec((T2, 2 * D), lambda i: (i, 0)),
        out_shape=jax.ShapeDtypeStruct((nsb * T2, 2 * D), jnp.float32),
    )(tab_t)


def kernel(word_ids, emb_wt):
    n_rows = word_ids.shape[0] * word_ids.shape[1]
    ids = word_ids.astype(jnp.int32)
    ids = (ids & ~(2 * T2 - 1)) | ((ids >> T2_LOG) & 1) | ((ids & (T2 - 1)) << 1)
    ids = ids.reshape(n_rows // C, C)
    tab = _pair_table(emb_wt).reshape(-1, D)
    out = _build_gather(n_rows)(ids, tab)
    return out.reshape(*word_ids.shape, D)
